# baseline (speedup 1.0000x reference)
import functools

import jax
import jax.numpy as jnp
from jax import lax
from jax.experimental import pallas as pl
from jax.experimental.pallas import tpu as pltpu
from jax.experimental.pallas import tpu_sc as plsc

VOCAB = 100000
EMBED = 32
BATCH = 16384
SEQ = 200

NC, NS = 2, 16
NW = NC * NS
NROWS = BATCH * SEQ
IW = 128
NIDXROWS = NROWS // IW
IDXROWS_PER_W = NIDXROWS // NW
CH = 10
ITERS = IDXROWS_PER_W // CH
PAIRS = ITERS // 2


@functools.partial(
    pl.kernel,
    out_type=jax.ShapeDtypeStruct((NIDXROWS, IW, EMBED), jnp.float32),
    mesh=plsc.VectorSubcoreMesh(core_axis_name="c", subcore_axis_name="s"),
    scratch_types=[
        pltpu.VMEM((CH, IW), jnp.int32),
        pltpu.VMEM((CH, IW), jnp.int32),
        pltpu.VMEM((CH, IW, EMBED), jnp.float32),
        pltpu.VMEM((CH, IW, EMBED), jnp.float32),
        pltpu.SemaphoreType.DMA,
        pltpu.SemaphoreType.DMA,
        pltpu.SemaphoreType.DMA,
        pltpu.SemaphoreType.DMA,
        pltpu.SemaphoreType.DMA,
        pltpu.SemaphoreType.DMA,
    ],
    compiler_params=pltpu.CompilerParams(use_tc_tiling_on_sc=False),
)
def _gather_all(tbl_hbm, idx_hbm, out_hbm,
                idx_v0, idx_v1, rows_v0, rows_v1,
                si0, si1, sg0, sg1, so0, so1):
    idx_v = (idx_v0, idx_v1)
    rows_v = (rows_v0, rows_v1)
    sem_i = (si0, si1)
    sem_g = (sg0, sg1)
    sem_o = (so0, so1)

    wid = lax.axis_index("s") * NC + lax.axis_index("c")
    row0 = wid * IDXROWS_PER_W

    def idx_src(g):
        return idx_hbm.at[pl.ds(row0 + g * CH, CH)]

    def out_dst(g):
        return out_hbm.at[pl.ds(row0 + g * CH, CH)]

    pltpu.async_copy(idx_src(0), idx_v[0], sem_i[0])

    def pair(p, carry):
        for s in (0, 1):
            g = p * 2 + s
            pltpu.make_async_copy(idx_src(g), idx_v[s], sem_i[s]).wait()

            @pl.when(g >= 2)
            def _():
                pltpu.make_async_copy(
                    rows_v[s], out_dst(g - 2), sem_o[s]).wait()

            for j in range(CH):
                pltpu.async_copy(
                    tbl_hbm.at[idx_v[s].at[j]], rows_v[s].at[j], sem_g[s])

            @pl.when(g >= 1)
            def _():
                pltpu.make_async_copy(
                    out_dst(g - 1), rows_v[1 - s], sem_g[1 - s]).wait()
                pltpu.async_copy(rows_v[1 - s], out_dst(g - 1), sem_o[1 - s])

            @pl.when(g + 1 < ITERS)
            def _():
                pltpu.async_copy(idx_src(g + 1), idx_v[1 - s], sem_i[1 - s])

        return carry

    lax.fori_loop(0, PAIRS, pair, 0)

    sl = (ITERS - 1) % 2
    pltpu.make_async_copy(out_dst(ITERS - 1), rows_v[sl], sem_g[sl]).wait()
    pltpu.async_copy(rows_v[sl], out_dst(ITERS - 1), sem_o[sl])
    pltpu.make_async_copy(rows_v[1 - sl], out_dst(ITERS - 2),
                          sem_o[1 - sl]).wait()
    pltpu.make_async_copy(rows_v[sl], out_dst(ITERS - 1), sem_o[sl]).wait()


def kernel(inchi, table, start_var):
    b, l = inchi.shape
    tok = inchi[:, :-1].astype(jnp.int32)
    idx = jnp.concatenate(
        [jnp.full((b, 1), VOCAB, jnp.int32), tok], axis=1)
    idx_rows = idx.reshape(NIDXROWS, IW)
    tbl = jnp.concatenate([table, start_var], axis=0)
    out = _gather_all(tbl, idx_rows)
    return out.reshape(b, l, EMBED)
